# Initial kernel scaffold; baseline (speedup 1.0000x reference)
import jax
import jax.numpy as jnp
from jax.experimental import pallas as pl


def kernel(inputs, table):
    raise NotImplementedError("write your pallas kernel here")


# speedup vs baseline: 247.1080x; 247.1080x over previous
import functools

import jax
import jax.numpy as jnp
from jax import lax
from jax.experimental import pallas as pl
from jax.experimental.pallas import tpu as pltpu
from jax.experimental.pallas import tpu_sc as plsc

_NC = 2
_NS = 16
_NW = _NC * _NS
_L = 16


@functools.lru_cache(maxsize=None)
def _sc_lookup(n_total: int, table_n: int, chunk: int):
    assert n_total % (_NW * chunk) == 0
    steps = n_total // (_NW * chunk)
    per_w = steps * chunk
    mesh = plsc.VectorSubcoreMesh(core_axis_name="c", subcore_axis_name="s")

    @functools.partial(
        pl.kernel,
        mesh=mesh,
        out_type=jax.ShapeDtypeStruct((n_total,), jnp.int32),
        scratch_types=[
            pltpu.VMEM((table_n,), jnp.int32),
            pltpu.VMEM((chunk,), jnp.int32),
        ],
        compiler_params=pltpu.CompilerParams(needs_layout_passes=False),
    )
    def k(idx_hbm, table_hbm, out_hbm, table_v, buf):
        wid = lax.axis_index("s") * _NC + lax.axis_index("c")
        pltpu.sync_copy(table_hbm, table_v)
        base0 = pl.multiple_of(wid * per_w, 8)

        def step(s, _):
            base = pl.multiple_of(base0 + s * chunk, 8)
            pltpu.sync_copy(idx_hbm.at[pl.ds(base, chunk)], buf)

            def body(i, _):
                off = i * _L
                idx = buf[pl.ds(off, _L)]
                buf[pl.ds(off, _L)] = plsc.load_gather(table_v, [idx])
                return 0

            lax.fori_loop(0, chunk // _L, body, 0, unroll=4)
            pltpu.sync_copy(buf, out_hbm.at[pl.ds(base, chunk)])
            return 0

        lax.fori_loop(0, steps, step, 0)

    return k


def kernel(inputs, table):
    shape = inputs.shape
    flat = inputs.reshape(-1).astype(jnp.int32)
    out = _sc_lookup(flat.size, table.shape[0], 25600)(flat, table)
    return out.reshape(shape)


# speedup vs baseline: 382.3125x; 1.5471x over previous
import functools

import jax
import jax.numpy as jnp
from jax import lax
from jax.experimental import pallas as pl
from jax.experimental.pallas import tpu as pltpu
from jax.experimental.pallas import tpu_sc as plsc

_NC = 2
_NS = 16
_NW = _NC * _NS
_L = 16


@functools.lru_cache(maxsize=None)
def _sc_lookup(n_rows: int, n_cols: int, table_n: int, rblk: int):
    assert n_rows % (_NW * rblk) == 0
    steps = n_rows // (_NW * rblk)
    rows_per_w = steps * rblk
    offs = list(range(0, n_cols - _L + 1, _L))
    if n_cols % _L:
        offs.append(n_cols - _L)
    mesh = plsc.VectorSubcoreMesh(core_axis_name="c", subcore_axis_name="s")

    @functools.partial(
        pl.kernel,
        mesh=mesh,
        out_type=jax.ShapeDtypeStruct((n_rows, n_cols), jnp.int32),
        scratch_types=[
            pltpu.VMEM((table_n,), jnp.int32),
            pltpu.VMEM((rblk, n_cols), jnp.int32),
            pltpu.VMEM((rblk, n_cols), jnp.int32),
        ],
        compiler_params=pltpu.CompilerParams(needs_layout_passes=False),
    )
    def k(idx_hbm, table_hbm, out_hbm, table_v, bin_v, bout_v):
        wid = lax.axis_index("s") * _NC + lax.axis_index("c")
        pltpu.sync_copy(table_hbm, table_v)
        row0 = pl.multiple_of(wid * rows_per_w, 8)

        def step(s, _):
            base = pl.multiple_of(row0 + s * rblk, 8)
            pltpu.sync_copy(idx_hbm.at[pl.ds(base, rblk)], bin_v)

            def body(r, _):
                for c in offs:
                    idx = bin_v[r, pl.ds(c, _L)]
                    bout_v[r, pl.ds(c, _L)] = plsc.load_gather(table_v, [idx])
                return 0

            lax.fori_loop(0, rblk, body, 0)
            pltpu.sync_copy(bout_v, out_hbm.at[pl.ds(base, rblk)])
            return 0

        lax.fori_loop(0, steps, step, 0)

    return k


def kernel(inputs, table):
    idx = inputs.astype(jnp.int32)
    return _sc_lookup(idx.shape[0], idx.shape[1], table.shape[0], 128)(idx, table)


# speedup vs baseline: 577.2198x; 1.5098x over previous
import functools

import jax
import jax.numpy as jnp
from jax import lax
from jax.experimental import pallas as pl
from jax.experimental.pallas import tpu as pltpu
from jax.experimental.pallas import tpu_sc as plsc

_NC = 2
_NS = 16
_NW = _NC * _NS
_L = 16


@functools.lru_cache(maxsize=None)
def _sc_lookup(n_rows: int, n_cols: int, table_n: int, cblk: int):
    assert n_cols % (_NW * cblk) == 0 and cblk % _L == 0
    steps = n_cols // (_NW * cblk)
    cols_per_w = steps * cblk
    vecs_per_row = cblk // _L
    mesh = plsc.VectorSubcoreMesh(core_axis_name="c", subcore_axis_name="s")

    @functools.partial(
        pl.kernel,
        mesh=mesh,
        out_type=jax.ShapeDtypeStruct((n_rows, n_cols), jnp.int32),
        scratch_types=[
            pltpu.VMEM((table_n,), jnp.int32),
            pltpu.VMEM((n_rows, cblk), jnp.int32),
            pltpu.VMEM((n_rows, cblk), jnp.int32),
        ],
        compiler_params=pltpu.CompilerParams(needs_layout_passes=False),
    )
    def k(idx_hbm, table_hbm, out_hbm, table_v, bin_v, bout_v):
        wid = lax.axis_index("s") * _NC + lax.axis_index("c")
        pltpu.sync_copy(table_hbm, table_v)
        col0 = pl.multiple_of(wid * cols_per_w, 8)

        def step(s, _):
            base = pl.multiple_of(col0 + s * cblk, 8)
            pltpu.sync_copy(idx_hbm.at[:, pl.ds(base, cblk)], bin_v)

            def body(r, _):
                for v in range(vecs_per_row):
                    idx = bin_v[r, pl.ds(v * _L, _L)]
                    bout_v[r, pl.ds(v * _L, _L)] = plsc.load_gather(
                        table_v, [idx]
                    )
                return 0

            lax.fori_loop(0, n_rows, body, 0)
            pltpu.sync_copy(bout_v, out_hbm.at[:, pl.ds(base, cblk)])
            return 0

        lax.fori_loop(0, steps, step, 0)

    return k


def kernel(inputs, table):
    idx_t = inputs.astype(jnp.int32).T
    out_t = _sc_lookup(idx_t.shape[0], idx_t.shape[1], table.shape[0], 128)(
        idx_t, table
    )
    return out_t.T
